# Initial kernel scaffold; baseline (speedup 1.0000x reference)
import jax
import jax.numpy as jnp
from jax.experimental import pallas as pl


def kernel(word_inputs, word_seq_lengths, word_embedding):
    raise NotImplementedError("write your pallas kernel here")


# speedup vs baseline: 1.4241x; 1.4241x over previous
import functools

import jax
import jax.numpy as jnp
from jax import lax
from jax.experimental import pallas as pl
from jax.experimental.pallas import tpu as pltpu
from jax.experimental.pallas import tpu_sc as plsc

BATCH = 4096
SEQ = 200
EMB_DIM = 32

NUM_CORES = 2
NUM_SUBCORES = 16
NW = NUM_CORES * NUM_SUBCORES

TOTAL = BATCH * SEQ
PER_W = TOTAL // NW
IDX_MINOR = 128
N_CHUNKS = PER_W // IDX_MINOR


def _gather_body(idx_hbm, table_hbm, out_hbm, idx_v, rows_v, gsem, osem):
    wid = lax.axis_index("s") * NUM_CORES + lax.axis_index("c")
    base = wid * PER_W
    pltpu.sync_copy(idx_hbm.at[wid], idx_v)

    def start_gather(j, slot):
        pltpu.async_copy(table_hbm.at[idx_v.at[j]], rows_v.at[slot], gsem)

    start_gather(0, 0)

    def body(j, _):
        slot = lax.rem(j, 2)
        nxt = lax.rem(j + 1, 2)

        @pl.when(j + 1 < N_CHUNKS)
        def _():
            @pl.when(j >= 1)
            def _():
                pltpu.make_async_copy(
                    rows_v.at[nxt], out_hbm.at[pl.ds(base, IDX_MINOR)], osem
                ).wait()
            start_gather(j + 1, nxt)

        pltpu.make_async_copy(
            table_hbm.at[idx_v.at[j]], rows_v.at[slot], gsem
        ).wait()
        pltpu.async_copy(
            rows_v.at[slot], out_hbm.at[pl.ds(base + j * IDX_MINOR, IDX_MINOR)], osem
        )
        return 0

    lax.fori_loop(0, N_CHUNKS, body, 0)
    pltpu.make_async_copy(
        rows_v.at[0], out_hbm.at[pl.ds(base, IDX_MINOR)], osem
    ).wait()
    pltpu.make_async_copy(
        rows_v.at[0], out_hbm.at[pl.ds(base, IDX_MINOR)], osem
    ).wait()


@jax.jit
def _gather(idx, table):
    mesh = plsc.VectorSubcoreMesh(core_axis_name="c", subcore_axis_name="s")
    kfn = functools.partial(
        pl.kernel,
        mesh=mesh,
        out_type=jax.ShapeDtypeStruct((TOTAL, EMB_DIM), jnp.float32),
        scratch_types=[
            pltpu.VMEM((N_CHUNKS, IDX_MINOR), jnp.int32),
            pltpu.VMEM((2, IDX_MINOR, EMB_DIM), jnp.float32),
            pltpu.SemaphoreType.DMA,
            pltpu.SemaphoreType.DMA,
        ],
        compiler_params=pltpu.CompilerParams(use_tc_tiling_on_sc=False),
    )(_gather_body)
    return kfn(idx, table)


def kernel(word_inputs, word_seq_lengths, word_embedding):
    del word_seq_lengths
    idx = word_inputs.reshape(NW, N_CHUNKS, IDX_MINOR).astype(jnp.int32)
    out = _gather(idx, word_embedding)
    return out.reshape(BATCH, SEQ, EMB_DIM)


# speedup vs baseline: 1.5027x; 1.0552x over previous
import functools

import jax
import jax.numpy as jnp
from jax import lax
from jax.experimental import pallas as pl
from jax.experimental.pallas import tpu as pltpu
from jax.experimental.pallas import tpu_sc as plsc

BATCH = 4096
SEQ = 200
EMB_DIM = 32

NUM_CORES = 2
NUM_SUBCORES = 16
NW = NUM_CORES * NUM_SUBCORES

TOTAL = BATCH * SEQ
PER_W = TOTAL // NW
IDX_MINOR = 128
N_CHUNKS = PER_W // IDX_MINOR


NBUF = 16
K_LEAD = 12
D_LAG = 4


def _gather_body(idx_hbm, table_hbm, out_hbm, idx_v, rows_v, gsem, osem):
    wid = lax.axis_index("s") * NUM_CORES + lax.axis_index("c")
    base = wid * PER_W
    pltpu.sync_copy(idx_hbm.at[wid], idx_v)

    for g in range(K_LEAD):
        pltpu.async_copy(table_hbm.at[idx_v.at[g]], rows_v.at[g], gsem)

    def wait_wb():
        pltpu.make_async_copy(
            rows_v.at[0], out_hbm.at[pl.ds(base, IDX_MINOR)], osem
        ).wait()

    def body(j, _):
        slot = lax.rem(j, NBUF)
        pltpu.make_async_copy(
            table_hbm.at[idx_v.at[j]], rows_v.at[slot], gsem
        ).wait()
        pltpu.async_copy(
            rows_v.at[slot], out_hbm.at[pl.ds(base + j * IDX_MINOR, IDX_MINOR)], osem
        )

        @pl.when(j >= D_LAG)
        def _():
            wait_wb()

        @pl.when(j + K_LEAD < N_CHUNKS)
        def _():
            pltpu.async_copy(
                table_hbm.at[idx_v.at[j + K_LEAD]],
                rows_v.at[lax.rem(j + K_LEAD, NBUF)],
                gsem,
            )
        return 0

    lax.fori_loop(0, N_CHUNKS, body, 0)
    for _ in range(D_LAG):
        wait_wb()


@jax.jit
def _gather(idx, table):
    mesh = plsc.VectorSubcoreMesh(core_axis_name="c", subcore_axis_name="s")
    kfn = functools.partial(
        pl.kernel,
        mesh=mesh,
        out_type=jax.ShapeDtypeStruct((TOTAL, EMB_DIM), jnp.float32),
        scratch_types=[
            pltpu.VMEM((N_CHUNKS, IDX_MINOR), jnp.int32),
            pltpu.VMEM((NBUF, IDX_MINOR, EMB_DIM), jnp.float32),
            pltpu.SemaphoreType.DMA,
            pltpu.SemaphoreType.DMA,
        ],
        compiler_params=pltpu.CompilerParams(use_tc_tiling_on_sc=False),
    )(_gather_body)
    return kfn(idx, table)


def kernel(word_inputs, word_seq_lengths, word_embedding):
    del word_seq_lengths
    idx = word_inputs.reshape(NW, N_CHUNKS, IDX_MINOR).astype(jnp.int32)
    out = _gather(idx, word_embedding)
    return out.reshape(BATCH, SEQ, EMB_DIM)
